# baseline (speedup 1.0000x reference)
import jax
import jax.numpy as jnp
from jax.experimental import pallas as pl
from jax.experimental.pallas import tpu as pltpu

N_NODES = 10000
NFEAT = 128
NHID = 32
N_CLUSTERS = 10
ALPHA = 0.2
BM = 400
GRID = N_NODES // BM


def _gcdec_body(x_ref, adj_ref, w_ref, b_ref, mu_ref, out_ref, q_ref, support_ref):
    i = pl.program_id(0)

    @pl.when(i == 0)
    def _():
        support_ref[:] = jnp.dot(
            x_ref[:], w_ref[:], preferred_element_type=jnp.float32
        )

    out_blk = (
        jnp.dot(adj_ref[:], support_ref[:], preferred_element_type=jnp.float32)
        + b_ref[:]
    )
    out_ref[:] = out_blk

    mu = mu_ref[:]
    cross = jax.lax.dot_general(
        out_blk, mu, (((1,), (1,)), ((), ())),
        preferred_element_type=jnp.float32,
    )
    d2 = (
        jnp.sum(out_blk * out_blk, axis=1, keepdims=True)
        + jnp.sum(mu * mu, axis=1, keepdims=True).reshape(1, N_CLUSTERS)
        - 2.0 * cross
    )
    q = 1.0 / (1.0 + d2 / ALPHA + 1e-08)
    q = q ** (ALPHA + 1.0) / 2.0
    q_ref[:] = q / jnp.sum(q, axis=1, keepdims=True)


def kernel(x, adj, W, b, mu):
    b2 = b.reshape(1, NHID)
    out, q = pl.pallas_call(
        _gcdec_body,
        grid=(GRID,),
        in_specs=[
            pl.BlockSpec((N_NODES, NFEAT), lambda i: (0, 0)),
            pl.BlockSpec((BM, N_NODES), lambda i: (i, 0)),
            pl.BlockSpec((NFEAT, NHID), lambda i: (0, 0)),
            pl.BlockSpec((1, NHID), lambda i: (0, 0)),
            pl.BlockSpec((N_CLUSTERS, NHID), lambda i: (0, 0)),
        ],
        out_specs=[
            pl.BlockSpec((BM, NHID), lambda i: (i, 0)),
            pl.BlockSpec((BM, N_CLUSTERS), lambda i: (i, 0)),
        ],
        out_shape=[
            jax.ShapeDtypeStruct((N_NODES, NHID), jnp.float32),
            jax.ShapeDtypeStruct((N_NODES, N_CLUSTERS), jnp.float32),
        ],
        scratch_shapes=[pltpu.VMEM((N_NODES, NHID), jnp.float32)],
    )(x, adj, W, b2, mu)
    return (out, q)
